# baseline (device time: 50256 ns/iter reference)
import jax
import jax.numpy as jnp
from jax import lax
from jax.experimental import pallas as pl
from jax.experimental.pallas import tpu as pltpu

Z = 4
R = 4
M = 1024
MS = 256
D = 1024

_MESH = pl.DeviceIdType.MESH


def kernel(partial, gamma):
    x = partial.reshape(Z, R, MS, D)
    g = gamma.reshape(1, D)

    def body(x_hbm, g_ref, out_ref,
             xloc, ownb, acc_ref, snd_ref, bx_ref, by_ref, bd_ref,
             rr_buf, lr_buf,
             load_sem, rr_send, rr_recv, lr_send, lr_recv,
             pb_send, pb_recv):
        my_x = lax.axis_index("x")
        my_y = lax.axis_index("y")
        my_z = lax.axis_index("z")
        r = 2 * my_x + my_y
        zp = jnp.minimum(my_z + 1, Z - 1)
        zm = jnp.maximum(my_z - 1, 0)

        ldma = pltpu.make_async_copy(x_hbm.at[:, r], xloc, load_sem)
        ldma.start()

        bsem = pltpu.get_barrier_semaphore()
        pl.semaphore_signal(bsem, inc=1, device_id=(1 - my_x, my_y, my_z),
                            device_id_type=_MESH)
        pl.semaphore_signal(bsem, inc=1, device_id=(my_x, 1 - my_y, my_z),
                            device_id_type=_MESH)

        @pl.when(my_z > 0)
        def _():
            pl.semaphore_signal(bsem, inc=1, device_id=(my_x, my_y, zm),
                                device_id_type=_MESH)

        @pl.when(my_z < Z - 1)
        def _():
            pl.semaphore_signal(bsem, inc=1, device_id=(my_x, my_y, zp),
                                device_id_type=_MESH)

        pl.semaphore_wait(bsem, 3)

        @pl.when((my_z > 0) & (my_z < Z - 1))
        def _():
            pl.semaphore_wait(bsem, 1)

        ldma.wait()
        ownb[...] = xloc[...].astype(jnp.bfloat16)

        def rr_desc(c, src=None):
            return pltpu.make_async_remote_copy(
                src_ref=rr_buf.at[c] if src is None else src,
                dst_ref=rr_buf.at[c],
                send_sem=rr_send.at[c], recv_sem=rr_recv.at[c],
                device_id=(my_x, my_y, zp), device_id_type=_MESH)

        def lr_desc(c, src=None):
            return pltpu.make_async_remote_copy(
                src_ref=lr_buf.at[c] if src is None else src,
                dst_ref=lr_buf.at[c],
                send_sem=lr_send.at[c], recv_sem=lr_recv.at[c],
                device_id=(my_x, my_y, zm), device_id_type=_MESH)

        def rw_step(c):
            @pl.when(my_z == 0)
            def _():
                rr_desc(c, src=ownb.at[c]).start()

            @pl.when((my_z >= 1) & (my_z < c))
            def _():
                rr_desc(c).wait_recv()
                rr_buf[c] = rr_buf[c] + ownb[c]
                rr_desc(c).start()

        def lw_step(c):
            @pl.when(my_z == Z - 1)
            def _():
                lr_desc(c, src=ownb.at[c]).start()

            @pl.when((my_z <= Z - 2) & (my_z > c))
            def _():
                lr_desc(c).wait_recv()
                lr_buf[c] = lr_buf[c] + ownb[c]
                lr_desc(c).start()

        @pl.when(my_z % 2 == 1)
        def _():
            rw_step(3); lw_step(0); rw_step(2); lw_step(1)
            rw_step(1); lw_step(2)

        @pl.when(my_z % 2 == 0)
        def _():
            lw_step(0); rw_step(3); lw_step(1); rw_step(2)
            lw_step(2); rw_step(1)

        for c in range(Z):
            if c >= 1:
                @pl.when(my_z == c)
                def _(c=c):
                    rr_desc(c).wait_recv()
            if c <= Z - 2:
                @pl.when(my_z == c)
                def _(c=c):
                    lr_desc(c).wait_recv()

            @pl.when(my_z == c)
            def _(c=c):
                acc = xloc[c]
                if c >= 1:
                    acc = acc + rr_buf[c].astype(jnp.float32)
                if c <= Z - 2:
                    acc = acc + lr_buf[c].astype(jnp.float32)
                acc_ref[...] = acc

        y = acc_ref[...]
        rms = jnp.sqrt(jnp.mean(y * y, axis=-1, keepdims=True) + 1e-6)
        normed = y / rms * g_ref[...]
        out_ref[pl.ds(r * MS, MS), :] = normed
        snd_ref[...] = normed.astype(jnp.bfloat16)

        s1x = pltpu.make_async_remote_copy(
            src_ref=snd_ref, dst_ref=bx_ref,
            send_sem=pb_send.at[0], recv_sem=pb_recv.at[0],
            device_id=(1 - my_x, my_y, my_z), device_id_type=_MESH)
        s1y = pltpu.make_async_remote_copy(
            src_ref=snd_ref, dst_ref=by_ref,
            send_sem=pb_send.at[1], recv_sem=pb_recv.at[1],
            device_id=(my_x, 1 - my_y, my_z), device_id_type=_MESH)
        s2 = pltpu.make_async_remote_copy(
            src_ref=bx_ref, dst_ref=bd_ref,
            send_sem=pb_send.at[2], recv_sem=pb_recv.at[2],
            device_id=(my_x, 1 - my_y, my_z), device_id_type=_MESH)
        s1x.start()
        s1y.start()
        s1x.wait_recv()
        s2.start()
        r_x = 2 * (1 - my_x) + my_y
        out_ref[pl.ds(r_x * MS, MS), :] = bx_ref[...].astype(jnp.float32)
        s1y.wait_recv()
        r_y = 2 * my_x + (1 - my_y)
        out_ref[pl.ds(r_y * MS, MS), :] = by_ref[...].astype(jnp.float32)
        s2.wait_recv()
        r_d = 2 * (1 - my_x) + (1 - my_y)
        out_ref[pl.ds(r_d * MS, MS), :] = bd_ref[...].astype(jnp.float32)

        s1x.wait_send()
        s1y.wait_send()
        s2.wait_send()
        for c in range(1, Z):
            @pl.when(my_z < c)
            def _(c=c):
                rr_desc(c).wait_send()
        for c in range(Z - 1):
            @pl.when(my_z > c)
            def _(c=c):
                lr_desc(c).wait_send()

    return pl.pallas_call(
        body,
        out_shape=jax.ShapeDtypeStruct((M, D), jnp.float32),
        in_specs=[
            pl.BlockSpec(memory_space=pl.ANY),
            pl.BlockSpec(memory_space=pltpu.VMEM),
        ],
        out_specs=pl.BlockSpec(memory_space=pltpu.VMEM),
        scratch_shapes=[
            pltpu.VMEM((Z, MS, D), jnp.float32),
            pltpu.VMEM((Z, MS, D), jnp.bfloat16),
            pltpu.VMEM((MS, D), jnp.float32),
            pltpu.VMEM((MS, D), jnp.bfloat16),
            pltpu.VMEM((MS, D), jnp.bfloat16),
            pltpu.VMEM((MS, D), jnp.bfloat16),
            pltpu.VMEM((MS, D), jnp.bfloat16),
            pltpu.VMEM((Z, MS, D), jnp.bfloat16),
            pltpu.VMEM((Z, MS, D), jnp.bfloat16),
            pltpu.SemaphoreType.DMA,
            pltpu.SemaphoreType.DMA((Z,)),
            pltpu.SemaphoreType.DMA((Z,)),
            pltpu.SemaphoreType.DMA((Z,)),
            pltpu.SemaphoreType.DMA((Z,)),
            pltpu.SemaphoreType.DMA((3,)),
            pltpu.SemaphoreType.DMA((3,)),
        ],
        compiler_params=pltpu.CompilerParams(collective_id=0),
    )(x, g)


# device time: 50137 ns/iter; 1.0024x vs baseline; 1.0024x over previous
import jax
import jax.numpy as jnp
from jax import lax
from jax.experimental import pallas as pl
from jax.experimental.pallas import tpu as pltpu

Z = 4
M = 1024
MS = 256
D = 1024

_MESH = pl.DeviceIdType.MESH


def kernel(partial, gamma):
    g = gamma.reshape(1, D)

    def body(x_hbm, g_ref, out_ref,
             xloc, acc_ref, snd_ref, bx_ref, by_ref, bd_ref,
             rr_buf, lr_buf,
             load_sems, rr_send, rr_recv, lr_send, lr_recv,
             pb_send, pb_recv):
        my_x = lax.axis_index("x")
        my_y = lax.axis_index("y")
        my_z = lax.axis_index("z")
        r = 2 * my_x + my_y
        zp = jnp.minimum(my_z + 1, Z - 1)
        zm = jnp.maximum(my_z - 1, 0)

        def ldma(c):
            return pltpu.make_async_copy(
                x_hbm.at[0, pl.ds(c * M + r * MS, MS), :],
                xloc.at[c], load_sems.at[c])

        for c in (3, 0, 2, 1):
            ldma(c).start()

        bsem = pltpu.get_barrier_semaphore()
        pl.semaphore_signal(bsem, inc=1, device_id=(1 - my_x, my_y, my_z),
                            device_id_type=_MESH)
        pl.semaphore_signal(bsem, inc=1, device_id=(my_x, 1 - my_y, my_z),
                            device_id_type=_MESH)

        @pl.when(my_z > 0)
        def _():
            pl.semaphore_signal(bsem, inc=1, device_id=(my_x, my_y, zm),
                                device_id_type=_MESH)

        @pl.when(my_z < Z - 1)
        def _():
            pl.semaphore_signal(bsem, inc=1, device_id=(my_x, my_y, zp),
                                device_id_type=_MESH)

        pl.semaphore_wait(bsem, 3)

        @pl.when((my_z > 0) & (my_z < Z - 1))
        def _():
            pl.semaphore_wait(bsem, 1)

        def rr_desc(c):
            return pltpu.make_async_remote_copy(
                src_ref=rr_buf.at[c], dst_ref=rr_buf.at[c],
                send_sem=rr_send.at[c], recv_sem=rr_recv.at[c],
                device_id=(my_x, my_y, zp), device_id_type=_MESH)

        def lr_desc(c):
            return pltpu.make_async_remote_copy(
                src_ref=lr_buf.at[c], dst_ref=lr_buf.at[c],
                send_sem=lr_send.at[c], recv_sem=lr_recv.at[c],
                device_id=(my_x, my_y, zm), device_id_type=_MESH)

        def rw_step(c):
            @pl.when(my_z == 0)
            def _():
                ldma(c).wait()
                rr_buf[c] = xloc[c].astype(jnp.bfloat16)
                rr_desc(c).start()

            @pl.when((my_z >= 1) & (my_z < c))
            def _():
                rr_desc(c).wait_recv()
                ldma(c).wait()
                rr_buf[c] = rr_buf[c] + xloc[c].astype(jnp.bfloat16)
                rr_desc(c).start()

        def lw_step(c):
            @pl.when(my_z == Z - 1)
            def _():
                ldma(c).wait()
                lr_buf[c] = xloc[c].astype(jnp.bfloat16)
                lr_desc(c).start()

            @pl.when((my_z <= Z - 2) & (my_z > c))
            def _():
                lr_desc(c).wait_recv()
                ldma(c).wait()
                lr_buf[c] = lr_buf[c] + xloc[c].astype(jnp.bfloat16)
                lr_desc(c).start()

        @pl.when(my_z % 2 == 1)
        def _():
            rw_step(3); lw_step(0); rw_step(2); lw_step(1)
            rw_step(1); lw_step(2)

        @pl.when(my_z % 2 == 0)
        def _():
            lw_step(0); rw_step(3); lw_step(1); rw_step(2)
            lw_step(2); rw_step(1)

        for c in range(Z):
            if c >= 1:
                @pl.when(my_z == c)
                def _(c=c):
                    rr_desc(c).wait_recv()
            if c <= Z - 2:
                @pl.when(my_z == c)
                def _(c=c):
                    lr_desc(c).wait_recv()

            @pl.when(my_z == c)
            def _(c=c):
                ldma(c).wait()
                acc = xloc[c]
                if c >= 1:
                    acc = acc + rr_buf[c].astype(jnp.float32)
                if c <= Z - 2:
                    acc = acc + lr_buf[c].astype(jnp.float32)
                acc_ref[...] = acc

        y = acc_ref[...]
        rms = jnp.sqrt(jnp.mean(y * y, axis=-1, keepdims=True) + 1e-6)
        normed = y / rms * g_ref[...]
        out_ref[pl.ds(r * MS, MS), :] = normed
        snd_ref[...] = normed.astype(jnp.bfloat16)

        s1x = pltpu.make_async_remote_copy(
            src_ref=snd_ref, dst_ref=bx_ref,
            send_sem=pb_send.at[0], recv_sem=pb_recv.at[0],
            device_id=(1 - my_x, my_y, my_z), device_id_type=_MESH)
        s1y = pltpu.make_async_remote_copy(
            src_ref=snd_ref, dst_ref=by_ref,
            send_sem=pb_send.at[1], recv_sem=pb_recv.at[1],
            device_id=(my_x, 1 - my_y, my_z), device_id_type=_MESH)
        s2 = pltpu.make_async_remote_copy(
            src_ref=bx_ref, dst_ref=bd_ref,
            send_sem=pb_send.at[2], recv_sem=pb_recv.at[2],
            device_id=(my_x, 1 - my_y, my_z), device_id_type=_MESH)
        s1x.start()
        s1y.start()
        s1x.wait_recv()
        s2.start()
        r_x = 2 * (1 - my_x) + my_y
        out_ref[pl.ds(r_x * MS, MS), :] = bx_ref[...].astype(jnp.float32)
        s1y.wait_recv()
        r_y = 2 * my_x + (1 - my_y)
        out_ref[pl.ds(r_y * MS, MS), :] = by_ref[...].astype(jnp.float32)
        s2.wait_recv()
        r_d = 2 * (1 - my_x) + (1 - my_y)
        out_ref[pl.ds(r_d * MS, MS), :] = bd_ref[...].astype(jnp.float32)

        s1x.wait_send()
        s1y.wait_send()
        s2.wait_send()
        for c in range(1, Z):
            @pl.when(my_z < c)
            def _(c=c):
                rr_desc(c).wait_send()
        for c in range(Z - 1):
            @pl.when(my_z > c)
            def _(c=c):
                lr_desc(c).wait_send()

    return pl.pallas_call(
        body,
        out_shape=jax.ShapeDtypeStruct((M, D), jnp.float32),
        in_specs=[
            pl.BlockSpec(memory_space=pl.ANY),
            pl.BlockSpec(memory_space=pltpu.VMEM),
        ],
        out_specs=pl.BlockSpec(memory_space=pltpu.VMEM),
        scratch_shapes=[
            pltpu.VMEM((Z, MS, D), jnp.float32),
            pltpu.VMEM((MS, D), jnp.float32),
            pltpu.VMEM((MS, D), jnp.bfloat16),
            pltpu.VMEM((MS, D), jnp.bfloat16),
            pltpu.VMEM((MS, D), jnp.bfloat16),
            pltpu.VMEM((MS, D), jnp.bfloat16),
            pltpu.VMEM((Z, MS, D), jnp.bfloat16),
            pltpu.VMEM((Z, MS, D), jnp.bfloat16),
            pltpu.SemaphoreType.DMA((Z,)),
            pltpu.SemaphoreType.DMA((Z,)),
            pltpu.SemaphoreType.DMA((Z,)),
            pltpu.SemaphoreType.DMA((Z,)),
            pltpu.SemaphoreType.DMA((Z,)),
            pltpu.SemaphoreType.DMA((3,)),
            pltpu.SemaphoreType.DMA((3,)),
        ],
        compiler_params=pltpu.CompilerParams(collective_id=0),
    )(partial, g)
